# baseline (speedup 1.0000x reference)
import functools

import jax
import jax.numpy as jnp
from jax import lax
from jax.experimental import pallas as pl
from jax.experimental.pallas import tpu as pltpu
from jax.experimental.pallas import tpu_sc as plsc

NC = 2
NS = 16
NW = NC * NS
CH = 128
DEG_W = 128


def _ceil_to(x, m):
    return (x + m - 1) // m * m


@functools.lru_cache(maxsize=None)
def _build_sc_kernels(n_nodes, n_edges, feat):
    n_pad = _ceil_to(n_nodes + 1, NS * 8)
    slab = n_pad // NS
    cpt = _ceil_to(-(-n_edges // (CH * NW)), 8)
    pages = cpt * NW
    e_pad = pages * CH

    mesh = plsc.VectorSubcoreMesh(core_axis_name="c", subcore_axis_name="s")

    @functools.partial(
        pl.kernel,
        out_type=jax.ShapeDtypeStruct((NC, n_pad, DEG_W), jnp.float32),
        mesh=mesh,
        scratch_types=[
            pltpu.VMEM((cpt, CH), jnp.int32),
            pltpu.VMEM((CH, DEG_W), jnp.float32),
            pltpu.VMEM_SHARED((n_pad, DEG_W), jnp.float32),
            pltpu.SemaphoreType.DMA,
        ],
    )
    def deg_kernel(dst_hbm, ones_hbm, zeros_hbm, out_hbm,
                   dst_v, ones_v, deg_sp, sem):
        c = lax.axis_index("c")
        s = lax.axis_index("s")
        wid = c * NS + s
        pltpu.sync_copy(zeros_hbm, deg_sp.at[pl.ds(s * slab, slab)])
        pltpu.sync_copy(ones_hbm, ones_v)
        pltpu.sync_copy(dst_hbm.at[pl.ds(wid * cpt, cpt)], dst_v)
        plsc.subcore_barrier()
        def body(j, carry):
            pltpu.sync_copy(ones_v, deg_sp.at[dst_v.at[j]], add=True)
            return carry
        lax.fori_loop(0, cpt, body, 0)
        plsc.subcore_barrier()
        pltpu.sync_copy(deg_sp.at[pl.ds(s * slab, slab)],
                        out_hbm.at[c].at[pl.ds(s * slab, slab)])

    cpt0 = _ceil_to(int(round(pages * 1.0)) // NS, 8)
    cpt1 = pages // NS - cpt0

    @functools.partial(
        pl.kernel,
        out_type=jax.ShapeDtypeStruct((NC, n_pad, feat), jnp.float32),
        mesh=mesh,
        scratch_types=[
            pltpu.VMEM((8, CH), jnp.int32),
            pltpu.VMEM((CH, feat), jnp.float32),
            pltpu.VMEM((CH, feat), jnp.float32),
            pltpu.VMEM_SHARED((n_pad, feat), jnp.float32),
            pltpu.SemaphoreType.DMA,
            pltpu.SemaphoreType.DMA,
        ],
    )
    def agg_kernel(y_hbm, comb_hbm, out_hbm,
                   idx_v, rows_a, rows_b, agg_sp, sem_a, sem_b):
        c = lax.axis_index("c")
        s = lax.axis_index("s")

        def zrow(i, carry):
            for j in range(feat // 16):
                rows_a[i, pl.ds(j * 16, 16)] = jnp.zeros((16,), jnp.float32)
            return carry
        lax.fori_loop(0, CH, zrow, 0)
        for off in range(0, slab - CH + 1, CH):
            pltpu.sync_copy(rows_a, agg_sp.at[pl.ds(s * slab + off, CH)])
        rem = slab % CH
        if rem:
            pltpu.sync_copy(rows_a.at[pl.ds(0, rem)],
                            agg_sp.at[pl.ds(s * slab + slab - rem, rem)])
        plsc.subcore_barrier()

        my_cpt = jnp.where(c == 0, cpt0, cpt1)
        base = c * NS * cpt0 + s * my_cpt

        def group(g, carry):
            row8 = 2 * (base + 4 * g)
            pltpu.sync_copy(comb_hbm.at[pl.ds(row8, 8)], idx_v)
            cp_a0 = pltpu.async_copy(y_hbm.at[idx_v.at[0]], rows_a, sem_a)
            cp_b1 = pltpu.async_copy(y_hbm.at[idx_v.at[1]], rows_b, sem_b)
            cp_a0.wait()
            pltpu.sync_copy(rows_a, agg_sp.at[idx_v.at[4]], add=True)
            cp_a2 = pltpu.async_copy(y_hbm.at[idx_v.at[2]], rows_a, sem_a)
            cp_b1.wait()
            pltpu.sync_copy(rows_b, agg_sp.at[idx_v.at[5]], add=True)
            cp_b3 = pltpu.async_copy(y_hbm.at[idx_v.at[3]], rows_b, sem_b)
            cp_a2.wait()
            pltpu.sync_copy(rows_a, agg_sp.at[idx_v.at[6]], add=True)
            cp_b3.wait()
            pltpu.sync_copy(rows_b, agg_sp.at[idx_v.at[7]], add=True)
            return carry
        lax.fori_loop(0, my_cpt // 4, group, 0)
        plsc.subcore_barrier()
        pltpu.sync_copy(agg_sp.at[pl.ds(s * slab, slab)],
                        out_hbm.at[c].at[pl.ds(s * slab, slab)])

    return deg_kernel, agg_kernel, n_pad, pages, e_pad


def _dis_from_deg(deg_ref):
    d = deg_ref[0, :, 0:1] + deg_ref[1, :, 0:1]
    return jnp.where(d > 0, lax.rsqrt(d), 0.0)


def _tc1_body(deg_ref, x_ref, w0_ref, b0_ref, w1_ref, h0_ref, y1_ref):
    dis = _dis_from_deg(deg_ref)
    h0 = jnp.maximum(
        jnp.dot(x_ref[...], w0_ref[...], preferred_element_type=jnp.float32)
        + b0_ref[...], 0.0)
    h0_ref[...] = h0
    y1_ref[...] = dis * jnp.dot(h0, w1_ref[...],
                                preferred_element_type=jnp.float32)


def _tc2_body(deg_ref, t_ref, b_ref, hin_ref, w_ref, h_ref, y_ref):
    dis = _dis_from_deg(deg_ref)
    agg = dis * (t_ref[0] + t_ref[1])
    h = jnp.maximum(agg + b_ref[...], 0.0) + hin_ref[...]
    h_ref[...] = h
    y_ref[...] = dis * jnp.dot(h, w_ref[...],
                               preferred_element_type=jnp.float32)


def _tc3_body(deg_ref, t_ref, b_ref, hin_ref, out_ref):
    dis = _dis_from_deg(deg_ref)
    agg = dis * (t_ref[0] + t_ref[1])
    out_ref[...] = jnp.maximum(agg + b_ref[...], 0.0) + hin_ref[...]


@functools.lru_cache(maxsize=None)
def _build_tc_kernels(n_nodes, n_pad, feat):
    rb = 1000 if n_nodes % 1000 == 0 else n_nodes
    grid = (n_nodes // rb,)
    deg_spec = pl.BlockSpec((NC, rb, DEG_W), lambda i: (0, i, 0))
    row_spec = pl.BlockSpec((rb, feat), lambda i: (i, 0))
    mat_spec = pl.BlockSpec((feat, feat), lambda i: (0, 0))
    bias_spec = pl.BlockSpec((1, feat), lambda i: (0, 0))
    t_spec = pl.BlockSpec((NC, rb, feat), lambda i: (0, i, 0))
    row_out = jax.ShapeDtypeStruct((n_nodes, feat), jnp.float32)

    tc1 = pl.pallas_call(
        _tc1_body, grid=grid,
        in_specs=[deg_spec, row_spec, mat_spec, bias_spec, mat_spec],
        out_specs=[row_spec, row_spec],
        out_shape=[row_out, row_out],
    )
    tc2 = pl.pallas_call(
        _tc2_body, grid=grid,
        in_specs=[deg_spec, t_spec, bias_spec, row_spec, mat_spec],
        out_specs=[row_spec, row_spec],
        out_shape=[row_out, row_out],
    )
    tc3 = pl.pallas_call(
        _tc3_body, grid=grid,
        in_specs=[deg_spec, t_spec, bias_spec, row_spec],
        out_specs=row_spec,
        out_shape=row_out,
    )
    return tc1, tc2, tc3


def kernel(features, edge_index, W0, b0, W1, b1, W2, b2):
    n, feat = features.shape
    e = edge_index.shape[1]
    deg_k, agg_k, n_pad, pages, e_pad = _build_sc_kernels(n, e, feat)
    tc1, tc2, tc3 = _build_tc_kernels(n, n_pad, feat)

    src = edge_index[0].astype(jnp.int32)
    dst = edge_index[1].astype(jnp.int32)
    pad = e_pad - e
    src_p = jnp.concatenate([src, jnp.zeros((pad,), jnp.int32)]).reshape(pages, CH)
    dst_p = jnp.concatenate([dst, jnp.full((pad,), n, jnp.int32)]).reshape(pages, CH)
    comb = jnp.concatenate(
        [src_p.reshape(-1, 4, CH), dst_p.reshape(-1, 4, CH)], axis=1
    ).reshape(2 * pages, CH)

    ones_rows = jnp.ones((CH, DEG_W), jnp.float32)
    zeros_deg = jnp.zeros((n_pad // NS, DEG_W), jnp.float32)

    deg_part = deg_k(dst_p, ones_rows, zeros_deg)

    b0r, b1r, b2r = (b.reshape(1, feat) for b in (b0, b1, b2))
    h0, y1 = tc1(deg_part, features, W0, b0r, W1)
    t1 = agg_k(y1, comb)
    h1, y2 = tc2(deg_part, t1, b1r, h0, W2)
    t2 = agg_k(y2, comb)
    return tc3(deg_part, t2, b2r, h1)


# speedup vs baseline: 2.7555x; 2.7555x over previous
import functools

import jax
import jax.numpy as jnp
from jax import lax
from jax.experimental import pallas as pl
from jax.experimental.pallas import tpu as pltpu
from jax.experimental.pallas import tpu_sc as plsc

NC = 2
NS = 16
NW = NC * NS
CH = 128
DEG_W = 128


def _ceil_to(x, m):
    return (x + m - 1) // m * m


@functools.lru_cache(maxsize=None)
def _build_sc_kernels(n_nodes, n_edges, feat):
    n_pad = _ceil_to(n_nodes + 1, NS * 8)
    slab = n_pad // NS
    cpt = _ceil_to(-(-n_edges // (CH * NW)), 8)
    pages = cpt * NW
    e_pad = pages * CH

    mesh = plsc.VectorSubcoreMesh(core_axis_name="c", subcore_axis_name="s")

    @functools.partial(
        pl.kernel,
        out_type=jax.ShapeDtypeStruct((NC, n_pad, DEG_W), jnp.float32),
        mesh=mesh,
        scratch_types=[
            pltpu.VMEM((cpt, CH), jnp.int32),
            pltpu.VMEM((CH, DEG_W), jnp.float32),
            pltpu.VMEM_SHARED((n_pad, DEG_W), jnp.float32),
            pltpu.SemaphoreType.DMA,
        ],
    )
    def deg_kernel(dst_hbm, ones_hbm, zeros_hbm, out_hbm,
                   dst_v, ones_v, deg_sp, sem):
        c = lax.axis_index("c")
        s = lax.axis_index("s")
        wid = c * NS + s
        pltpu.sync_copy(zeros_hbm, deg_sp.at[pl.ds(s * slab, slab)])
        pltpu.sync_copy(ones_hbm, ones_v)
        pltpu.sync_copy(dst_hbm.at[pl.ds(wid * cpt, cpt)], dst_v)
        plsc.subcore_barrier()
        def body(j, carry):
            pltpu.sync_copy(ones_v, deg_sp.at[dst_v.at[j]], add=True)
            return carry
        lax.fori_loop(0, cpt, body, 0)
        plsc.subcore_barrier()
        pltpu.sync_copy(deg_sp.at[pl.ds(s * slab, slab)],
                        out_hbm.at[c].at[pl.ds(s * slab, slab)])

    cpt0 = _ceil_to(int(round(pages * 0.5)) // NS, 8)
    cpt1 = pages // NS - cpt0

    @functools.partial(
        pl.kernel,
        out_type=jax.ShapeDtypeStruct((NC, n_pad, feat), jnp.float32),
        mesh=mesh,
        scratch_types=[
            pltpu.VMEM((8, CH), jnp.int32),
            pltpu.VMEM((CH, feat), jnp.float32),
            pltpu.VMEM((CH, feat), jnp.float32),
            pltpu.VMEM_SHARED((n_pad, feat), jnp.float32),
            pltpu.SemaphoreType.DMA,
            pltpu.SemaphoreType.DMA,
        ],
    )
    def agg_kernel(y_hbm, comb_hbm, out_hbm,
                   idx_v, rows_a, rows_b, agg_sp, sem_a, sem_b):
        c = lax.axis_index("c")
        s = lax.axis_index("s")

        def zrow(i, carry):
            for j in range(feat // 16):
                rows_a[i, pl.ds(j * 16, 16)] = jnp.zeros((16,), jnp.float32)
            return carry
        lax.fori_loop(0, CH, zrow, 0)
        for off in range(0, slab - CH + 1, CH):
            pltpu.sync_copy(rows_a, agg_sp.at[pl.ds(s * slab + off, CH)])
        rem = slab % CH
        if rem:
            pltpu.sync_copy(rows_a.at[pl.ds(0, rem)],
                            agg_sp.at[pl.ds(s * slab + slab - rem, rem)])
        plsc.subcore_barrier()

        my_cpt = jnp.where(c == 0, cpt0, cpt1)
        base = c * NS * cpt0 + s * my_cpt

        def group(g, carry):
            row8 = 2 * (base + 4 * g)
            pltpu.sync_copy(comb_hbm.at[pl.ds(row8, 8)], idx_v)
            cp_a0 = pltpu.async_copy(y_hbm.at[idx_v.at[0]], rows_a, sem_a)
            cp_b1 = pltpu.async_copy(y_hbm.at[idx_v.at[1]], rows_b, sem_b)
            cp_a0.wait()
            pltpu.sync_copy(rows_a, agg_sp.at[idx_v.at[4]], add=True)
            cp_a2 = pltpu.async_copy(y_hbm.at[idx_v.at[2]], rows_a, sem_a)
            cp_b1.wait()
            pltpu.sync_copy(rows_b, agg_sp.at[idx_v.at[5]], add=True)
            cp_b3 = pltpu.async_copy(y_hbm.at[idx_v.at[3]], rows_b, sem_b)
            cp_a2.wait()
            pltpu.sync_copy(rows_a, agg_sp.at[idx_v.at[6]], add=True)
            cp_b3.wait()
            pltpu.sync_copy(rows_b, agg_sp.at[idx_v.at[7]], add=True)
            return carry
        lax.fori_loop(0, my_cpt // 4, group, 0)
        plsc.subcore_barrier()
        pltpu.sync_copy(agg_sp.at[pl.ds(s * slab, slab)],
                        out_hbm.at[c].at[pl.ds(s * slab, slab)])

    return deg_kernel, agg_kernel, n_pad, pages, e_pad


def _dis_from_deg(deg_ref):
    d = deg_ref[0, :, 0:1] + deg_ref[1, :, 0:1]
    return jnp.where(d > 0, lax.rsqrt(d), 0.0)


def _tc1_body(deg_ref, x_ref, w0_ref, b0_ref, w1_ref, h0_ref, y1_ref):
    dis = _dis_from_deg(deg_ref)
    h0 = jnp.maximum(
        jnp.dot(x_ref[...], w0_ref[...], preferred_element_type=jnp.float32)
        + b0_ref[...], 0.0)
    h0_ref[...] = h0
    y1_ref[...] = dis * jnp.dot(h0, w1_ref[...],
                                preferred_element_type=jnp.float32)


def _tc2_body(deg_ref, t_ref, b_ref, hin_ref, w_ref, h_ref, y_ref):
    dis = _dis_from_deg(deg_ref)
    agg = dis * (t_ref[0] + t_ref[1])
    h = jnp.maximum(agg + b_ref[...], 0.0) + hin_ref[...]
    h_ref[...] = h
    y_ref[...] = dis * jnp.dot(h, w_ref[...],
                               preferred_element_type=jnp.float32)


def _tc3_body(deg_ref, t_ref, b_ref, hin_ref, out_ref):
    dis = _dis_from_deg(deg_ref)
    agg = dis * (t_ref[0] + t_ref[1])
    out_ref[...] = jnp.maximum(agg + b_ref[...], 0.0) + hin_ref[...]


@functools.lru_cache(maxsize=None)
def _build_tc_kernels(n_nodes, n_pad, feat):
    rb = 1000 if n_nodes % 1000 == 0 else n_nodes
    grid = (n_nodes // rb,)
    deg_spec = pl.BlockSpec((NC, rb, DEG_W), lambda i: (0, i, 0))
    row_spec = pl.BlockSpec((rb, feat), lambda i: (i, 0))
    mat_spec = pl.BlockSpec((feat, feat), lambda i: (0, 0))
    bias_spec = pl.BlockSpec((1, feat), lambda i: (0, 0))
    t_spec = pl.BlockSpec((NC, rb, feat), lambda i: (0, i, 0))
    row_out = jax.ShapeDtypeStruct((n_nodes, feat), jnp.float32)

    tc1 = pl.pallas_call(
        _tc1_body, grid=grid,
        in_specs=[deg_spec, row_spec, mat_spec, bias_spec, mat_spec],
        out_specs=[row_spec, row_spec],
        out_shape=[row_out, row_out],
    )
    tc2 = pl.pallas_call(
        _tc2_body, grid=grid,
        in_specs=[deg_spec, t_spec, bias_spec, row_spec, mat_spec],
        out_specs=[row_spec, row_spec],
        out_shape=[row_out, row_out],
    )
    tc3 = pl.pallas_call(
        _tc3_body, grid=grid,
        in_specs=[deg_spec, t_spec, bias_spec, row_spec],
        out_specs=row_spec,
        out_shape=row_out,
    )
    return tc1, tc2, tc3


def kernel(features, edge_index, W0, b0, W1, b1, W2, b2):
    n, feat = features.shape
    e = edge_index.shape[1]
    deg_k, agg_k, n_pad, pages, e_pad = _build_sc_kernels(n, e, feat)
    tc1, tc2, tc3 = _build_tc_kernels(n, n_pad, feat)

    src = edge_index[0].astype(jnp.int32)
    dst = edge_index[1].astype(jnp.int32)
    pad = e_pad - e
    pad_src = jnp.arange(pad, dtype=jnp.int32) % n
    pad_dst = n + jnp.arange(pad, dtype=jnp.int32) % (n_pad - n)
    src_p = jnp.concatenate([src, pad_src]).reshape(pages, CH)
    dst_p = jnp.concatenate([dst, pad_dst]).reshape(pages, CH)
    comb = jnp.concatenate(
        [src_p.reshape(-1, 4, CH), dst_p.reshape(-1, 4, CH)], axis=1
    ).reshape(2 * pages, CH)

    ones_rows = jnp.ones((CH, DEG_W), jnp.float32)
    zeros_deg = jnp.zeros((n_pad // NS, DEG_W), jnp.float32)

    deg_part = deg_k(dst_p, ones_rows, zeros_deg)

    b0r, b1r, b2r = (b.reshape(1, feat) for b in (b0, b1, b2))
    h0, y1 = tc1(deg_part, features, W0, b0r, W1)
    t1 = agg_k(y1, comb)
    h1, y2 = tc2(deg_part, t1, b1r, h0, W2)
    t2 = agg_k(y2, comb)
    return tc3(deg_part, t2, b2r, h1)


# speedup vs baseline: 2.9143x; 1.0576x over previous
import functools

import jax
import jax.numpy as jnp
from jax import lax
from jax.experimental import pallas as pl
from jax.experimental.pallas import tpu as pltpu
from jax.experimental.pallas import tpu_sc as plsc

NC = 2
NS = 16
NW = NC * NS
CH = 128
DEG_W = 128


def _ceil_to(x, m):
    return (x + m - 1) // m * m


@functools.lru_cache(maxsize=None)
def _build_sc_kernels(n_nodes, n_edges, feat):
    n_pad = _ceil_to(n_nodes + 1, NS * 8)
    slab = n_pad // NS
    cpt = _ceil_to(-(-n_edges // (CH * NW)), 8)
    pages = cpt * NW
    e_pad = pages * CH

    mesh = plsc.VectorSubcoreMesh(core_axis_name="c", subcore_axis_name="s")

    @functools.partial(
        pl.kernel,
        out_type=jax.ShapeDtypeStruct((NC, n_pad, DEG_W), jnp.float32),
        mesh=mesh,
        scratch_types=[
            pltpu.VMEM((cpt, CH), jnp.int32),
            pltpu.VMEM((CH, DEG_W), jnp.float32),
            pltpu.VMEM_SHARED((n_pad, DEG_W), jnp.float32),
            pltpu.SemaphoreType.DMA,
        ],
    )
    def deg_kernel(dst_hbm, ones_hbm, zeros_hbm, out_hbm,
                   dst_v, ones_v, deg_sp, sem):
        c = lax.axis_index("c")
        s = lax.axis_index("s")
        wid = c * NS + s
        pltpu.sync_copy(zeros_hbm, deg_sp.at[pl.ds(s * slab, slab)])
        pltpu.sync_copy(ones_hbm, ones_v)
        pltpu.sync_copy(dst_hbm.at[pl.ds(wid * cpt, cpt)], dst_v)
        plsc.subcore_barrier()
        def body(j, carry):
            pltpu.sync_copy(ones_v, deg_sp.at[dst_v.at[j]], add=True)
            return carry
        lax.fori_loop(0, cpt, body, 0)
        plsc.subcore_barrier()
        pltpu.sync_copy(deg_sp.at[pl.ds(s * slab, slab)],
                        out_hbm.at[c].at[pl.ds(s * slab, slab)])

    cpt0 = _ceil_to(int(round(pages * 0.5)) // NS, 8)
    cpt1 = pages // NS - cpt0

    @functools.partial(
        pl.kernel,
        out_type=jax.ShapeDtypeStruct((NC, n_pad, feat), jnp.float32),
        mesh=mesh,
        scratch_types=[
            pltpu.VMEM((2, 8, CH), jnp.int32),
            pltpu.VMEM((CH, feat), jnp.float32),
            pltpu.VMEM((CH, feat), jnp.float32),
            pltpu.VMEM_SHARED((n_pad, feat), jnp.float32),
            pltpu.SemaphoreType.DMA,
            pltpu.SemaphoreType.DMA,
            pltpu.SemaphoreType.DMA,
        ],
    )
    def agg_kernel(y_hbm, comb_hbm, out_hbm,
                   idx_v, rows_a, rows_b, agg_sp, sem_a, sem_b, sem_i):
        c = lax.axis_index("c")
        s = lax.axis_index("s")

        def zrow(i, carry):
            for j in range(feat // 16):
                rows_a[i, pl.ds(j * 16, 16)] = jnp.zeros((16,), jnp.float32)
            return carry
        lax.fori_loop(0, CH, zrow, 0)
        for off in range(0, slab - CH + 1, CH):
            pltpu.sync_copy(rows_a, agg_sp.at[pl.ds(s * slab + off, CH)])
        rem = slab % CH
        if rem:
            pltpu.sync_copy(rows_a.at[pl.ds(0, rem)],
                            agg_sp.at[pl.ds(s * slab + slab - rem, rem)])
        plsc.subcore_barrier()

        my_cpt = jnp.where(c == 0, cpt0, cpt1)
        base = c * NS * cpt0 + s * my_cpt
        n_groups = my_cpt // 4
        last_row8 = 2 * (base + 4 * (n_groups - 1))

        pltpu.async_copy(comb_hbm.at[pl.ds(2 * base, 8)], idx_v.at[0], sem_i)

        def group(g, carry):
            cur = idx_v.at[lax.rem(g, 2)]
            nxt = idx_v.at[lax.rem(g + 1, 2)]
            row8 = 2 * (base + 4 * g)
            pltpu.make_async_copy(comb_hbm.at[pl.ds(row8, 8)], cur,
                                  sem_i).wait()
            row8n = jnp.minimum(row8 + 8, last_row8)
            pltpu.async_copy(comb_hbm.at[pl.ds(row8n, 8)], nxt, sem_i)
            cp_a0 = pltpu.async_copy(y_hbm.at[cur.at[0]], rows_a, sem_a)
            cp_b1 = pltpu.async_copy(y_hbm.at[cur.at[1]], rows_b, sem_b)
            cp_a0.wait()
            pltpu.sync_copy(rows_a, agg_sp.at[cur.at[4]], add=True)
            cp_a2 = pltpu.async_copy(y_hbm.at[cur.at[2]], rows_a, sem_a)
            cp_b1.wait()
            pltpu.sync_copy(rows_b, agg_sp.at[cur.at[5]], add=True)
            cp_b3 = pltpu.async_copy(y_hbm.at[cur.at[3]], rows_b, sem_b)
            cp_a2.wait()
            pltpu.sync_copy(rows_a, agg_sp.at[cur.at[6]], add=True)
            cp_b3.wait()
            pltpu.sync_copy(rows_b, agg_sp.at[cur.at[7]], add=True)
            return carry
        lax.fori_loop(0, n_groups, group, 0)
        pltpu.make_async_copy(comb_hbm.at[pl.ds(2 * base, 8)],
                              idx_v.at[lax.rem(n_groups, 2)], sem_i).wait()
        plsc.subcore_barrier()
        pltpu.sync_copy(agg_sp.at[pl.ds(s * slab, slab)],
                        out_hbm.at[c].at[pl.ds(s * slab, slab)])

    return deg_kernel, agg_kernel, n_pad, pages, e_pad


def _dis_from_deg(deg_ref):
    d = deg_ref[0, :, 0:1] + deg_ref[1, :, 0:1]
    return jnp.where(d > 0, lax.rsqrt(d), 0.0)


def _tc1a_body(x_ref, w0_ref, b0_ref, h0_ref):
    h0_ref[...] = jnp.maximum(
        jnp.dot(x_ref[...], w0_ref[...], preferred_element_type=jnp.float32)
        + b0_ref[...], 0.0)


def _tc1b_body(deg_ref, h0_ref, w1_ref, y1_ref):
    dis = _dis_from_deg(deg_ref)
    y1_ref[...] = dis * jnp.dot(h0_ref[...], w1_ref[...],
                                preferred_element_type=jnp.float32)


def _tc2_body(deg_ref, t_ref, b_ref, hin_ref, w_ref, h_ref, y_ref):
    dis = _dis_from_deg(deg_ref)
    agg = dis * (t_ref[0] + t_ref[1])
    h = jnp.maximum(agg + b_ref[...], 0.0) + hin_ref[...]
    h_ref[...] = h
    y_ref[...] = dis * jnp.dot(h, w_ref[...],
                               preferred_element_type=jnp.float32)


def _tc3_body(deg_ref, t_ref, b_ref, hin_ref, out_ref):
    dis = _dis_from_deg(deg_ref)
    agg = dis * (t_ref[0] + t_ref[1])
    out_ref[...] = jnp.maximum(agg + b_ref[...], 0.0) + hin_ref[...]


@functools.lru_cache(maxsize=None)
def _build_tc_kernels(n_nodes, n_pad, feat):
    rb = 1000 if n_nodes % 1000 == 0 else n_nodes
    grid = (n_nodes // rb,)
    deg_spec = pl.BlockSpec((NC, rb, DEG_W), lambda i: (0, i, 0))
    row_spec = pl.BlockSpec((rb, feat), lambda i: (i, 0))
    mat_spec = pl.BlockSpec((feat, feat), lambda i: (0, 0))
    bias_spec = pl.BlockSpec((1, feat), lambda i: (0, 0))
    t_spec = pl.BlockSpec((NC, rb, feat), lambda i: (0, i, 0))
    row_out = jax.ShapeDtypeStruct((n_nodes, feat), jnp.float32)

    tc1a = pl.pallas_call(
        _tc1a_body, grid=grid,
        in_specs=[row_spec, mat_spec, bias_spec],
        out_specs=row_spec,
        out_shape=row_out,
    )
    tc1b = pl.pallas_call(
        _tc1b_body, grid=grid,
        in_specs=[deg_spec, row_spec, mat_spec],
        out_specs=row_spec,
        out_shape=row_out,
    )
    tc2 = pl.pallas_call(
        _tc2_body, grid=grid,
        in_specs=[deg_spec, t_spec, bias_spec, row_spec, mat_spec],
        out_specs=[row_spec, row_spec],
        out_shape=[row_out, row_out],
    )
    tc3 = pl.pallas_call(
        _tc3_body, grid=grid,
        in_specs=[deg_spec, t_spec, bias_spec, row_spec],
        out_specs=row_spec,
        out_shape=row_out,
    )
    return tc1a, tc1b, tc2, tc3


def kernel(features, edge_index, W0, b0, W1, b1, W2, b2):
    n, feat = features.shape
    e = edge_index.shape[1]
    deg_k, agg_k, n_pad, pages, e_pad = _build_sc_kernels(n, e, feat)
    tc1a, tc1b, tc2, tc3 = _build_tc_kernels(n, n_pad, feat)

    src = edge_index[0].astype(jnp.int32)
    dst = edge_index[1].astype(jnp.int32)
    pad = e_pad - e
    pad_src = jnp.arange(pad, dtype=jnp.int32) % n
    pad_dst = n + jnp.arange(pad, dtype=jnp.int32) % (n_pad - n)
    src_p = jnp.concatenate([src, pad_src]).reshape(pages, CH)
    dst_p = jnp.concatenate([dst, pad_dst]).reshape(pages, CH)
    comb = jnp.concatenate(
        [src_p.reshape(-1, 4, CH), dst_p.reshape(-1, 4, CH)], axis=1
    ).reshape(2 * pages, CH)

    ones_rows = jnp.ones((CH, DEG_W), jnp.float32)
    zeros_deg = jnp.zeros((n_pad // NS, DEG_W), jnp.float32)

    deg_part = deg_k(dst_p, ones_rows, zeros_deg)

    b0r, b1r, b2r = (b.reshape(1, feat) for b in (b0, b1, b2))
    h0 = tc1a(features, W0, b0r)
    y1 = tc1b(deg_part, h0, W1)
    t1 = agg_k(y1, comb)
    h1, y2 = tc2(deg_part, t1, b1r, h0, W2)
    t2 = agg_k(y2, comb)
    return tc3(deg_part, t2, b2r, h1)


# speedup vs baseline: 2.9174x; 1.0011x over previous
import functools

import jax
import jax.numpy as jnp
from jax import lax
from jax.experimental import pallas as pl
from jax.experimental.pallas import tpu as pltpu
from jax.experimental.pallas import tpu_sc as plsc

NC = 2
NS = 16
NW = NC * NS
CH = 128
DEG_W = 128


def _ceil_to(x, m):
    return (x + m - 1) // m * m


@functools.lru_cache(maxsize=None)
def _build_sc_kernels(n_nodes, n_edges, feat):
    n_pad = _ceil_to(n_nodes + 1, NS * 8)
    slab = n_pad // NS
    cpt = _ceil_to(-(-n_edges // (CH * NW)), 8)
    pages = cpt * NW
    e_pad = pages * CH

    mesh = plsc.VectorSubcoreMesh(core_axis_name="c", subcore_axis_name="s")

    @functools.partial(
        pl.kernel,
        out_type=jax.ShapeDtypeStruct((NC, n_pad, DEG_W), jnp.float32),
        mesh=mesh,
        scratch_types=[
            pltpu.VMEM((cpt, CH), jnp.int32),
            pltpu.VMEM((CH, DEG_W), jnp.float32),
            pltpu.VMEM_SHARED((n_pad, DEG_W), jnp.float32),
            pltpu.SemaphoreType.DMA,
        ],
    )
    def deg_kernel(dst_hbm, ones_hbm, zeros_hbm, out_hbm,
                   dst_v, ones_v, deg_sp, sem):
        c = lax.axis_index("c")
        s = lax.axis_index("s")
        wid = c * NS + s
        pltpu.sync_copy(zeros_hbm, deg_sp.at[pl.ds(s * slab, slab)])
        pltpu.sync_copy(ones_hbm, ones_v)
        pltpu.sync_copy(dst_hbm.at[pl.ds(wid * cpt, cpt)], dst_v)
        plsc.subcore_barrier()
        def body(j, carry):
            pltpu.async_copy(ones_v, deg_sp.at[dst_v.at[j]], sem, add=True)
            return carry
        lax.fori_loop(0, cpt, body, 0)
        def drain(j, carry):
            pltpu.make_async_copy(ones_v, deg_sp.at[dst_v.at[0]], sem).wait()
            return carry
        lax.fori_loop(0, cpt, drain, 0)
        plsc.subcore_barrier()
        pltpu.sync_copy(deg_sp.at[pl.ds(s * slab, slab)],
                        out_hbm.at[c].at[pl.ds(s * slab, slab)])

    cpt0 = _ceil_to(int(round(pages * 0.5)) // NS, 8)
    cpt1 = pages // NS - cpt0

    @functools.partial(
        pl.kernel,
        out_type=jax.ShapeDtypeStruct((NC, n_pad, feat), jnp.float32),
        mesh=mesh,
        scratch_types=[
            pltpu.VMEM((2, 8, CH), jnp.int32),
            pltpu.VMEM((CH, feat), jnp.float32),
            pltpu.VMEM((CH, feat), jnp.float32),
            pltpu.VMEM_SHARED((n_pad, feat), jnp.float32),
            pltpu.SemaphoreType.DMA,
            pltpu.SemaphoreType.DMA,
            pltpu.SemaphoreType.DMA,
        ],
    )
    def agg_kernel(y_hbm, comb_hbm, out_hbm,
                   idx_v, rows_a, rows_b, agg_sp, sem_a, sem_b, sem_i):
        c = lax.axis_index("c")
        s = lax.axis_index("s")

        def zrow(i, carry):
            for j in range(feat // 16):
                rows_a[i, pl.ds(j * 16, 16)] = jnp.zeros((16,), jnp.float32)
            return carry
        lax.fori_loop(0, CH, zrow, 0)
        for off in range(0, slab - CH + 1, CH):
            pltpu.sync_copy(rows_a, agg_sp.at[pl.ds(s * slab + off, CH)])
        rem = slab % CH
        if rem:
            pltpu.sync_copy(rows_a.at[pl.ds(0, rem)],
                            agg_sp.at[pl.ds(s * slab + slab - rem, rem)])
        plsc.subcore_barrier()

        my_cpt = jnp.where(c == 0, cpt0, cpt1)
        base = c * NS * cpt0 + s * my_cpt
        n_groups = my_cpt // 4
        last_row8 = 2 * (base + 4 * (n_groups - 1))

        pltpu.async_copy(comb_hbm.at[pl.ds(2 * base, 8)], idx_v.at[0], sem_i)

        def group(g, carry):
            cur = idx_v.at[lax.rem(g, 2)]
            nxt = idx_v.at[lax.rem(g + 1, 2)]
            row8 = 2 * (base + 4 * g)
            pltpu.make_async_copy(comb_hbm.at[pl.ds(row8, 8)], cur,
                                  sem_i).wait()
            row8n = jnp.minimum(row8 + 8, last_row8)
            pltpu.async_copy(comb_hbm.at[pl.ds(row8n, 8)], nxt, sem_i)
            cp_a0 = pltpu.async_copy(y_hbm.at[cur.at[0]], rows_a, sem_a)
            cp_b1 = pltpu.async_copy(y_hbm.at[cur.at[1]], rows_b, sem_b)
            cp_a0.wait()
            pltpu.sync_copy(rows_a, agg_sp.at[cur.at[4]], add=True)
            cp_a2 = pltpu.async_copy(y_hbm.at[cur.at[2]], rows_a, sem_a)
            cp_b1.wait()
            pltpu.sync_copy(rows_b, agg_sp.at[cur.at[5]], add=True)
            cp_b3 = pltpu.async_copy(y_hbm.at[cur.at[3]], rows_b, sem_b)
            cp_a2.wait()
            pltpu.sync_copy(rows_a, agg_sp.at[cur.at[6]], add=True)
            cp_b3.wait()
            pltpu.sync_copy(rows_b, agg_sp.at[cur.at[7]], add=True)
            return carry
        lax.fori_loop(0, n_groups, group, 0)
        pltpu.make_async_copy(comb_hbm.at[pl.ds(2 * base, 8)],
                              idx_v.at[lax.rem(n_groups, 2)], sem_i).wait()
        plsc.subcore_barrier()
        pltpu.sync_copy(agg_sp.at[pl.ds(s * slab, slab)],
                        out_hbm.at[c].at[pl.ds(s * slab, slab)])

    return deg_kernel, agg_kernel, n_pad, pages, e_pad


def _dis_from_deg(deg_ref):
    d = deg_ref[0, :, 0:1] + deg_ref[1, :, 0:1]
    return jnp.where(d > 0, lax.rsqrt(d), 0.0)


def _tc1a_body(x_ref, w0_ref, b0_ref, h0_ref):
    h0_ref[...] = jnp.maximum(
        jnp.dot(x_ref[...], w0_ref[...], preferred_element_type=jnp.float32)
        + b0_ref[...], 0.0)


def _tc1b_body(deg_ref, h0_ref, w1_ref, y1_ref):
    dis = _dis_from_deg(deg_ref)
    y1_ref[...] = dis * jnp.dot(h0_ref[...], w1_ref[...],
                                preferred_element_type=jnp.float32)


def _tc2_body(deg_ref, t_ref, b_ref, hin_ref, w_ref, h_ref, y_ref):
    dis = _dis_from_deg(deg_ref)
    agg = dis * (t_ref[0] + t_ref[1])
    h = jnp.maximum(agg + b_ref[...], 0.0) + hin_ref[...]
    h_ref[...] = h
    y_ref[...] = dis * jnp.dot(h, w_ref[...],
                               preferred_element_type=jnp.float32)


def _tc3_body(deg_ref, t_ref, b_ref, hin_ref, out_ref):
    dis = _dis_from_deg(deg_ref)
    agg = dis * (t_ref[0] + t_ref[1])
    out_ref[...] = jnp.maximum(agg + b_ref[...], 0.0) + hin_ref[...]


@functools.lru_cache(maxsize=None)
def _build_tc_kernels(n_nodes, n_pad, feat):
    rb = 1000 if n_nodes % 1000 == 0 else n_nodes
    grid = (n_nodes // rb,)
    deg_spec = pl.BlockSpec((NC, rb, DEG_W), lambda i: (0, i, 0))
    row_spec = pl.BlockSpec((rb, feat), lambda i: (i, 0))
    mat_spec = pl.BlockSpec((feat, feat), lambda i: (0, 0))
    bias_spec = pl.BlockSpec((1, feat), lambda i: (0, 0))
    t_spec = pl.BlockSpec((NC, rb, feat), lambda i: (0, i, 0))
    row_out = jax.ShapeDtypeStruct((n_nodes, feat), jnp.float32)

    tc1a = pl.pallas_call(
        _tc1a_body, grid=grid,
        in_specs=[row_spec, mat_spec, bias_spec],
        out_specs=row_spec,
        out_shape=row_out,
    )
    tc1b = pl.pallas_call(
        _tc1b_body, grid=grid,
        in_specs=[deg_spec, row_spec, mat_spec],
        out_specs=row_spec,
        out_shape=row_out,
    )
    tc2 = pl.pallas_call(
        _tc2_body, grid=grid,
        in_specs=[deg_spec, t_spec, bias_spec, row_spec, mat_spec],
        out_specs=[row_spec, row_spec],
        out_shape=[row_out, row_out],
    )
    tc3 = pl.pallas_call(
        _tc3_body, grid=grid,
        in_specs=[deg_spec, t_spec, bias_spec, row_spec],
        out_specs=row_spec,
        out_shape=row_out,
    )
    return tc1a, tc1b, tc2, tc3


def kernel(features, edge_index, W0, b0, W1, b1, W2, b2):
    n, feat = features.shape
    e = edge_index.shape[1]
    deg_k, agg_k, n_pad, pages, e_pad = _build_sc_kernels(n, e, feat)
    tc1a, tc1b, tc2, tc3 = _build_tc_kernels(n, n_pad, feat)

    src = edge_index[0].astype(jnp.int32)
    dst = edge_index[1].astype(jnp.int32)
    pad = e_pad - e
    pad_src = jnp.arange(pad, dtype=jnp.int32) % n
    pad_dst = n + jnp.arange(pad, dtype=jnp.int32) % (n_pad - n)
    src_p = jnp.concatenate([src, pad_src]).reshape(pages, CH)
    dst_p = jnp.concatenate([dst, pad_dst]).reshape(pages, CH)
    comb = jnp.concatenate(
        [src_p.reshape(-1, 4, CH), dst_p.reshape(-1, 4, CH)], axis=1
    ).reshape(2 * pages, CH)

    ones_rows = jnp.ones((CH, DEG_W), jnp.float32)
    zeros_deg = jnp.zeros((n_pad // NS, DEG_W), jnp.float32)

    deg_part = deg_k(dst_p, ones_rows, zeros_deg)

    b0r, b1r, b2r = (b.reshape(1, feat) for b in (b0, b1, b2))
    h0 = tc1a(features, W0, b0r)
    y1 = tc1b(deg_part, h0, W1)
    t1 = agg_k(y1, comb)
    h1, y2 = tc2(deg_part, t1, b1r, h0, W2)
    t2 = agg_k(y2, comb)
    return tc3(deg_part, t2, b2r, h1)
